# Initial kernel scaffold; baseline (speedup 1.0000x reference)
import jax
import jax.numpy as jnp
from jax.experimental import pallas as pl


def kernel(x, expert_probs, expert_weights, expert_biases):
    raise NotImplementedError("write your pallas kernel here")


# speedup vs baseline: 4.8133x; 4.8133x over previous
import functools

import jax
import jax.numpy as jnp
from jax.experimental import pallas as pl
from jax.experimental.pallas import tpu as pltpu

B, S, D_IN, D_OUT, E = 4, 2048, 1024, 1024, 64

BS = 1024
BO = 1024


def _argmax_kernel(probs_ref, idx_ref):
    probs = probs_ref[...]
    idx_ref[...] = jnp.argmax(probs, axis=-1).astype(jnp.int32)[:, None]


def _moe_matmul_kernel(idx_ref, x_ref, w_ref, b_ref, out_ref):
    x = x_ref[0]
    w = w_ref[0]
    acc = jax.lax.dot_general(
        x, w,
        dimension_numbers=(((1,), (1,)), ((), ())),
        preferred_element_type=jnp.float32,
    )
    out_ref[0] = acc + b_ref[0]


def kernel(x, expert_probs, expert_weights, expert_biases):
    idx2d = pl.pallas_call(
        _argmax_kernel,
        out_shape=jax.ShapeDtypeStruct((B, 1), jnp.int32),
    )(expert_probs)
    expert_idx = idx2d.reshape(B)

    grid = (B, S // BS, D_OUT // BO)
    x_out = pl.pallas_call(
        _moe_matmul_kernel,
        grid_spec=pltpu.PrefetchScalarGridSpec(
            num_scalar_prefetch=1,
            grid=grid,
            in_specs=[
                pl.BlockSpec((1, BS, D_IN), lambda b, i, j, idx: (b, i, 0)),
                pl.BlockSpec((1, BO, D_IN), lambda b, i, j, idx: (idx[b], j, 0)),
                pl.BlockSpec((1, 1, BO), lambda b, i, j, idx: (idx[b], 0, j)),
            ],
            out_specs=pl.BlockSpec((1, BS, BO), lambda b, i, j, idx: (b, i, j)),
        ),
        out_shape=jax.ShapeDtypeStruct((B, S, D_OUT), jnp.float32),
    )(expert_idx, x, expert_weights, expert_biases.reshape(E, 1, D_OUT))

    return (x_out, expert_idx)


# speedup vs baseline: 5.3647x; 1.1146x over previous
import functools

import jax
import jax.numpy as jnp
from jax.experimental import pallas as pl
from jax.experimental.pallas import tpu as pltpu

B, S, D_IN, D_OUT, E = 4, 2048, 1024, 1024, 64

BS = 2048
BO = 1024


def _argmax_kernel(probs_ref, idx_ref):
    probs = probs_ref[...]
    idx_ref[...] = jnp.argmax(probs, axis=-1).astype(jnp.int32)[:, None]


def _moe_matmul_kernel(idx_ref, x_ref, w_ref, b_ref, out_ref):
    x = x_ref[0]
    w = w_ref[0]
    acc = jax.lax.dot_general(
        x, w,
        dimension_numbers=(((1,), (1,)), ((), ())),
        preferred_element_type=jnp.float32,
    )
    out_ref[0] = acc + b_ref[0]


def kernel(x, expert_probs, expert_weights, expert_biases):
    idx2d = pl.pallas_call(
        _argmax_kernel,
        out_shape=jax.ShapeDtypeStruct((B, 1), jnp.int32),
    )(expert_probs)
    expert_idx = idx2d.reshape(B)

    grid = (B, S // BS, D_OUT // BO)
    x_out = pl.pallas_call(
        _moe_matmul_kernel,
        grid_spec=pltpu.PrefetchScalarGridSpec(
            num_scalar_prefetch=1,
            grid=grid,
            in_specs=[
                pl.BlockSpec((1, BS, D_IN), lambda b, i, j, idx: (b, i, 0)),
                pl.BlockSpec((1, BO, D_IN), lambda b, i, j, idx: (idx[b], j, 0)),
                pl.BlockSpec((1, 1, BO), lambda b, i, j, idx: (idx[b], 0, j)),
            ],
            out_specs=pl.BlockSpec((1, BS, BO), lambda b, i, j, idx: (b, i, j)),
        ),
        out_shape=jax.ShapeDtypeStruct((B, S, D_OUT), jnp.float32),
    )(expert_idx, x, expert_weights, expert_biases.reshape(E, 1, D_OUT))

    return (x_out, expert_idx)
